# baseline (device time: 548211 ns/iter reference)
import os

import jax
import jax.numpy as jnp
from jax import lax
from jax.experimental import pallas as pl
from jax.experimental.pallas import tpu as pltpu

N_DEV = 4
_PROBE = os.environ.get("PROBE", "")

NT = 128
NTILES = 8192 // NT


def _fused_body(x_ref, w_ref, sx_ref, sw_ref, out_ref,
                xg, wv, wt, ob, ao, csems, sems, wt_sems, ob_sems, ao_sems):
    my = lax.axis_index("i")
    left = (my - 1) % N_DEV
    right = (my + 1) % N_DEV
    diag = (my + 2) % N_DEV
    m, kl = x_ref.shape
    mh = m // 2
    kh = kl // 2

    barrier = pltpu.get_barrier_semaphore()
    for nbr in (left, right):
        pl.semaphore_signal(barrier, inc=1, device_id=(nbr,),
                            device_id_type=pl.DeviceIdType.MESH)
    pl.semaphore_wait(barrier, 2)

    def rdma(src, dst, s_i, r_i, dev):
        return pltpu.make_async_remote_copy(
            src_ref=src, dst_ref=dst,
            send_sem=sems.at[s_i], recv_sem=sems.at[r_i],
            device_id=(dev,), device_id_type=pl.DeviceIdType.MESH)

    cx = pltpu.make_async_copy(x_ref, xg.at[my], csems.at[0])
    cx.start()
    p1 = []
    si = 0
    for dev, wslot in ((right, 0), (left, 1)):
        for half in (0, 1):
            p1.append(rdma(x_ref.at[pl.ds(half * mh, mh)],
                           xg.at[my, pl.ds(half * mh, mh)],
                           si, si + 1, dev))
            p1.append(rdma(w_ref.at[pl.ds(half * kh, kh)],
                           wv.at[wslot, pl.ds(half * kh, kh)],
                           si + 2, si + 3, dev))
            si += 4
    if _PROBE != "compute":
        for d in p1:
            d.start()

    s = sx_ref[0] * sw_ref[0]

    def run_pass(chunks, get_w, issue_w, accumulate, epilogue):
        n_s = len(chunks)

        def issue(nt):
            b = lax.rem(nt, 2)
            if issue_w is not None:
                issue_w(nt, b)
            if accumulate:
                pltpu.make_async_copy(
                    out_ref.at[:, pl.ds(nt * NT, NT)], ob.at[b],
                    ob_sems.at[b]).start()

        issue(0)

        def body(nt, carry):
            b = lax.rem(nt, 2)

            @pl.when(nt < NTILES - 1)
            def _():
                issue(nt + 1)

            acc = jnp.dot(xg[chunks[0]], get_w(0, nt, b),
                          preferred_element_type=jnp.float32)
            for ci in range(1, n_s):
                acc = acc + jnp.dot(xg[chunks[ci]], get_w(ci, nt, b),
                                    preferred_element_type=jnp.float32)
            if accumulate:
                pltpu.make_async_copy(
                    out_ref.at[:, pl.ds(nt * NT, NT)], ob.at[b],
                    ob_sems.at[b]).wait()
                acc = acc + ob[b]
            if epilogue:
                acc = jnp.maximum(acc * s, 0.0)

            @pl.when(nt >= 2)
            def _():
                pltpu.make_async_copy(
                    ao.at[b], out_ref.at[:, pl.ds((nt - 2) * NT, NT)],
                    ao_sems.at[b]).wait()

            ao[b] = acc
            pltpu.make_async_copy(
                ao.at[b], out_ref.at[:, pl.ds(nt * NT, NT)],
                ao_sems.at[b]).start()
            return carry

        lax.fori_loop(0, NTILES, body, 0)
        pltpu.make_async_copy(
            ao.at[0], out_ref.at[:, pl.ds((NTILES - 2) * NT, NT)],
            ao_sems.at[0]).wait()
        pltpu.make_async_copy(
            ao.at[1], out_ref.at[:, pl.ds((NTILES - 1) * NT, NT)],
            ao_sems.at[1]).wait()

    def issue_w0(nt, b):
        pltpu.make_async_copy(w_ref.at[:, pl.ds(nt * NT, NT)], wt.at[b],
                              wt_sems.at[b]).start()

    def get_w0(ci, nt, b):
        pltpu.make_async_copy(w_ref.at[:, pl.ds(nt * NT, NT)], wt.at[b],
                              wt_sems.at[b]).wait()
        return wt[b]

    cx.wait()
    if _PROBE not in ("comm", "p1"):
        run_pass([my], get_w0, issue_w0, accumulate=False, epilogue=False)

    if _PROBE != "compute":
        for d in p1:
            d.wait()

    p2 = [
        rdma(xg.at[right, pl.ds(0, mh)], xg.at[right, pl.ds(0, mh)],
             16, 17, left),
        rdma(wv.at[1, pl.ds(0, kh)], wv.at[2, pl.ds(0, kh)], 18, 19, left),
        rdma(xg.at[left, pl.ds(mh, mh)], xg.at[left, pl.ds(mh, mh)],
             20, 21, right),
        rdma(wv.at[0, pl.ds(kh, kh)], wv.at[2, pl.ds(kh, kh)], 22, 23, right),
    ]
    if _PROBE not in ("compute", "p1"):
        for d in p2:
            d.start()

    if _PROBE not in ("comm", "p1"):
        run_pass([left, right],
                 lambda ci, nt, b: wv[ci, :, pl.ds(nt * NT, NT)], None,
                 accumulate=True, epilogue=False)

    if _PROBE not in ("compute", "p1"):
        for d in p2:
            d.wait()

    if _PROBE not in ("comm", "p1"):
        run_pass([diag], lambda ci, nt, b: wv[2, :, pl.ds(nt * NT, NT)],
                 None, accumulate=True, epilogue=True)


def _fused(x8, w8, scale_x, scale_w):
    m, kl = x8.shape
    _, n = w8.shape
    return pl.pallas_call(
        _fused_body,
        out_shape=jax.ShapeDtypeStruct((m, n), jnp.float32),
        in_specs=[
            pl.BlockSpec(memory_space=pl.ANY),
            pl.BlockSpec(memory_space=pl.ANY),
            pl.BlockSpec(memory_space=pltpu.MemorySpace.SMEM),
            pl.BlockSpec(memory_space=pltpu.MemorySpace.SMEM),
        ],
        out_specs=pl.BlockSpec(memory_space=pl.ANY),
        scratch_shapes=[
            pltpu.VMEM((N_DEV, m, kl), x8.dtype),
            pltpu.VMEM((3, kl, n), x8.dtype),
            pltpu.VMEM((2, kl, NT), x8.dtype),
            pltpu.VMEM((2, m, NT), jnp.float32),
            pltpu.VMEM((2, m, NT), jnp.float32),
            pltpu.SemaphoreType.DMA((2,)),
            pltpu.SemaphoreType.DMA((24,)),
            pltpu.SemaphoreType.DMA((2,)),
            pltpu.SemaphoreType.DMA((2,)),
            pltpu.SemaphoreType.DMA((2,)),
        ],
        compiler_params=pltpu.CompilerParams(
            collective_id=0, vmem_limit_bytes=64 * 1024 * 1024),
    )(x8, w8, scale_x, scale_w)


def kernel(x, w_mat, scale_x, scale_w):
    x8 = x.astype(jnp.float8_e4m3fn)
    w8 = w_mat.astype(jnp.float8_e4m3fn)
    return _fused(x8, w8, scale_x, scale_w)
